# baseline (speedup 1.0000x reference)
import functools

import jax
import jax.numpy as jnp
from jax import lax
from jax.experimental import pallas as pl
from jax.experimental.pallas import tpu as pltpu
from jax.experimental.pallas import tpu_sc as plsc

N = 10000
NP = 10240
E = 320000
D = 128
F1 = 16
F2 = 32
G = 4
W8 = 16

NC = 2
NS = 16
NW = NC * NS
EPT = E // NW
B1 = 2000
NB1 = EPT // B1
B2 = 1000
NB2 = EPT // B2
NPT = NP // NS

_mesh = plsc.VectorSubcoreMesh(
    core_axis_name="c", subcore_axis_name="s", num_cores=NC, num_subcores=NS)
_sc_params = pltpu.CompilerParams(use_tc_tiling_on_sc=False)


@functools.partial(
    pl.kernel,
    out_type=jax.ShapeDtypeStruct((NC, NP, W8), jnp.float32),
    mesh=_mesh,
    scratch_types=[
        pltpu.VMEM_SHARED((NP, W8), jnp.float32),
        pltpu.VMEM((NB1, B1), jnp.int32),
        pltpu.VMEM((B1, W8), jnp.float32),
        pltpu.SemaphoreType.DMA,
        pltpu.SemaphoreType.DMA,
        pltpu.SemaphoreType.DMA,
        pltpu.SemaphoreType.DMA,
    ],
    compiler_params=_sc_params,
)
def _deg_kernel(dst_hbm, ones_hbm, zer_hbm, out_hbm, acc, idx_d, ones_v,
                sem_z, sem_i, sem_o, sem_sc):
    c = lax.axis_index("c")
    s = lax.axis_index("s")
    w = s * NC + c
    sl = pl.ds(s * NPT, NPT)
    zd = pltpu.async_copy(zer_hbm.at[sl], acc.at[sl], sem_z)
    di = pltpu.async_copy(dst_hbm.at[w], idx_d, sem_i)
    od = pltpu.async_copy(ones_hbm, ones_v, sem_o)
    zd.wait()
    di.wait()
    od.wait()
    plsc.subcore_barrier()
    descs = []
    for j in range(NB1):
        descs.append(
            pltpu.async_copy(ones_v, acc.at[idx_d.at[j]], sem_sc, add=True))
    for d in descs:
        d.wait()
    plsc.subcore_barrier()
    pltpu.sync_copy(acc.at[sl], out_hbm.at[c, sl])


def _make_scatter(F, Bk, NBk):
    @functools.partial(
        pl.kernel,
        out_type=jax.ShapeDtypeStruct((NC, NP, F), jnp.float32),
        mesh=_mesh,
        scratch_types=[
            pltpu.VMEM_SHARED((NP, F), jnp.float32),
            pltpu.VMEM((NBk, Bk), jnp.int32),
            pltpu.VMEM((NBk, Bk), jnp.int32),
            pltpu.VMEM((Bk, F), jnp.float32),
            pltpu.VMEM((Bk, F), jnp.float32),
            pltpu.SemaphoreType.DMA,
            pltpu.SemaphoreType.DMA,
            pltpu.SemaphoreType.DMA,
            pltpu.SemaphoreType.DMA,
            pltpu.SemaphoreType.DMA,
            pltpu.SemaphoreType.DMA,
        ],
        compiler_params=_sc_params,
    )
    def _scatter(g_hbm, src_hbm, dst_hbm, zer_hbm, out_hbm, acc, idx_s, idx_d,
                 rows0, rows1, sem_z, sem_is, sem_id, sem_g, sem_sc0, sem_sc1):
        c = lax.axis_index("c")
        s = lax.axis_index("s")
        w = s * NC + c
        sl = pl.ds(s * NPT, NPT)
        zd = pltpu.async_copy(zer_hbm.at[sl], acc.at[sl], sem_z)
        sd = pltpu.async_copy(src_hbm.at[w], idx_s, sem_is)
        dd = pltpu.async_copy(dst_hbm.at[w], idx_d, sem_id)
        zd.wait()
        sd.wait()
        dd.wait()
        plsc.subcore_barrier()
        rows = (rows0, rows1)
        sem_sc = (sem_sc0, sem_sc1)
        scs = [None] * NBk
        for j in range(NBk):
            if j >= 2:
                scs[j - 2].wait()
            gd = pltpu.async_copy(g_hbm.at[idx_s.at[j]], rows[j % 2], sem_g)
            gd.wait()
            scs[j] = pltpu.async_copy(rows[j % 2], acc.at[idx_d.at[j]],
                                      sem_sc[j % 2], add=True)
        scs[NBk - 1].wait()
        scs[NBk - 2].wait()
        plsc.subcore_barrier()
        pltpu.sync_copy(acc.at[sl], out_hbm.at[c, sl])

    return _scatter


_scatter16 = _make_scatter(F1, B1, NB1)
_scatter32 = _make_scatter(F2, B2, NB2)

RB = 1280
NRB = NP // RB


def _tc0_body(x_ref, w1_ref, hw_ref):
    hw_ref[...] = jnp.dot(x_ref[...], w1_ref[...],
                          preferred_element_type=jnp.float32,
                          precision=lax.Precision.HIGHEST)


def _tc0(xp, W1):
    return pl.pallas_call(
        _tc0_body,
        grid=(NRB,),
        in_specs=[
            pl.BlockSpec((RB, D), lambda i: (i, 0)),
            pl.BlockSpec((D, F1), lambda i: (0, 0)),
        ],
        out_specs=pl.BlockSpec((RB, F1), lambda i: (i, 0)),
        out_shape=jax.ShapeDtypeStruct((NP, F1), jnp.float32),
    )(xp, W1)


def _tc1_body(hw_ref, d0_ref, d1_ref, g1_ref, dinv_ref):
    deg = d0_ref[...] + d1_ref[...] + 1.0
    dinv = lax.rsqrt(deg)
    g1_ref[...] = hw_ref[...] * dinv
    dinv_ref[...] = dinv


def _tc1(hw, d0, d1):
    return pl.pallas_call(
        _tc1_body,
        grid=(NRB,),
        in_specs=[
            pl.BlockSpec((RB, F1), lambda i: (i, 0)),
            pl.BlockSpec((RB, 1), lambda i: (i, 0)),
            pl.BlockSpec((RB, 1), lambda i: (i, 0)),
        ],
        out_specs=[
            pl.BlockSpec((RB, F1), lambda i: (i, 0)),
            pl.BlockSpec((RB, 1), lambda i: (i, 0)),
        ],
        out_shape=[
            jax.ShapeDtypeStruct((NP, F1), jnp.float32),
            jax.ShapeDtypeStruct((NP, 1), jnp.float32),
        ],
    )(hw, d0, d1)


def _tc2_body(sp0_ref, sp1_ref, g1_ref, dinv_ref, b1_ref, w2_ref, g2_ref):
    dv = dinv_ref[...]
    h1 = jnp.maximum(
        dv * (sp0_ref[...] + sp1_ref[...] + g1_ref[...]) + b1_ref[...], 0.0)
    g2_ref[...] = jnp.dot(h1, w2_ref[...],
                          preferred_element_type=jnp.float32,
                          precision=lax.Precision.HIGHEST) * dv


def _tc2(sp0, sp1, g1, dinv, b1r, W2):
    return pl.pallas_call(
        _tc2_body,
        grid=(NRB,),
        in_specs=[
            pl.BlockSpec((RB, F1), lambda i: (i, 0)),
            pl.BlockSpec((RB, F1), lambda i: (i, 0)),
            pl.BlockSpec((RB, F1), lambda i: (i, 0)),
            pl.BlockSpec((RB, 1), lambda i: (i, 0)),
            pl.BlockSpec((1, F1), lambda i: (0, 0)),
            pl.BlockSpec((F1, F2), lambda i: (0, 0)),
        ],
        out_specs=pl.BlockSpec((RB, F2), lambda i: (i, 0)),
        out_shape=jax.ShapeDtypeStruct((NP, F2), jnp.float32),
    )(sp0, sp1, g1, dinv, b1r, W2)


def _tc3_body(tp0_ref, tp1_ref, g2_ref, dinv_ref, b2_ref, bi_ref, out_ref,
              cnt_ref):
    i = pl.program_id(0)

    @pl.when(i == 0)
    def _():
        out_ref[...] = jnp.zeros_like(out_ref)
        cnt_ref[...] = jnp.zeros_like(cnt_ref)

    h2 = jnp.maximum(
        dinv_ref[...] * (tp0_ref[...] + tp1_ref[...] + g2_ref[...])
        + b2_ref[...], 0.0)
    b = bi_ref[...]
    for g in range(G):
        m = b == g
        out_ref[g:g + 1, :] += jnp.sum(jnp.where(m, h2, 0.0), axis=0,
                                       keepdims=True)
        cnt_ref[g:g + 1, :] += jnp.sum(jnp.where(m, 1.0, 0.0), axis=0,
                                       keepdims=True)

    @pl.when(i == NRB - 1)
    def _():
        out_ref[...] = out_ref[...] / jnp.maximum(cnt_ref[...], 1.0)


def _tc3(tp0, tp1, g2, dinv, b2r, bip):
    return pl.pallas_call(
        _tc3_body,
        grid=(NRB,),
        in_specs=[
            pl.BlockSpec((RB, F2), lambda i: (i, 0)),
            pl.BlockSpec((RB, F2), lambda i: (i, 0)),
            pl.BlockSpec((RB, F2), lambda i: (i, 0)),
            pl.BlockSpec((RB, 1), lambda i: (i, 0)),
            pl.BlockSpec((1, F2), lambda i: (0, 0)),
            pl.BlockSpec((RB, 1), lambda i: (i, 0)),
        ],
        out_specs=pl.BlockSpec((G, F2), lambda i: (0, 0)),
        out_shape=jax.ShapeDtypeStruct((G, F2), jnp.float32),
        scratch_shapes=[pltpu.VMEM((G, 1), jnp.float32)],
    )(tp0, tp1, g2, dinv, b2r, bip)


def kernel(x, edge_index, batch_index, W1, b1, W2, b2):
    x = x.astype(jnp.float32)
    src = edge_index[0].astype(jnp.int32)
    dst = edge_index[1].astype(jnp.int32)
    bi = batch_index.astype(jnp.int32)

    xp = jnp.pad(x, ((0, NP - N), (0, 0)))
    bip = jnp.pad(bi, (0, NP - N), constant_values=G).reshape(NP, 1)

    src1 = src.reshape(NW, NB1, B1)
    dst1 = dst.reshape(NW, NB1, B1)
    src2 = src.reshape(NW, NB2, B2)
    dst2 = dst.reshape(NW, NB2, B2)
    ones8 = jnp.ones((B1, W8), jnp.float32)
    z8 = jnp.zeros((NP, W8), jnp.float32)
    z16 = jnp.zeros((NP, F1), jnp.float32)
    z32 = jnp.zeros((NP, F2), jnp.float32)

    hw = _tc0(xp, W1)
    degp = _deg_kernel(dst1, ones8, z8)
    d0 = degp[0, :, :1]
    d1 = degp[1, :, :1]
    g1, dinv = _tc1(hw, d0, d1)
    sp = _scatter16(g1, src1, dst1, z16)
    g2 = _tc2(sp[0], sp[1], g1, dinv, b1.reshape(1, F1), W2)
    tp = _scatter32(g2, src2, dst2, z32)
    pooled = _tc3(tp[0], tp[1], g2, dinv, b2.reshape(1, F2), bip)
    return pooled


# speedup vs baseline: 1.0407x; 1.0407x over previous
import functools

import jax
import jax.numpy as jnp
from jax import lax
from jax.experimental import pallas as pl
from jax.experimental.pallas import tpu as pltpu
from jax.experimental.pallas import tpu_sc as plsc

N = 10000
NP = 10240
E = 320000
D = 128
F1 = 16
F2 = 32
G = 4
W8 = 16

NC = 2
NS = 16
NW = NC * NS
EPT = E // NW
B1 = 2000
NB1 = EPT // B1
B2 = 1000
NB2 = EPT // B2
NPT = NP // NS

_mesh = plsc.VectorSubcoreMesh(
    core_axis_name="c", subcore_axis_name="s", num_cores=NC, num_subcores=NS)
_sc_params = pltpu.CompilerParams(use_tc_tiling_on_sc=False)


@functools.partial(
    pl.kernel,
    out_type=jax.ShapeDtypeStruct((NC, NP, W8), jnp.float32),
    mesh=_mesh,
    scratch_types=[
        pltpu.VMEM_SHARED((NP, W8), jnp.float32),
        pltpu.VMEM((NB1, B1), jnp.int32),
        pltpu.VMEM((B1, W8), jnp.float32),
        pltpu.SemaphoreType.DMA,
        pltpu.SemaphoreType.DMA,
        pltpu.SemaphoreType.DMA,
        pltpu.SemaphoreType.DMA,
    ],
    compiler_params=_sc_params,
)
def _deg_kernel(dst_hbm, ones_hbm, zer_hbm, out_hbm, acc, idx_d, ones_v,
                sem_z, sem_i, sem_o, sem_sc):
    c = lax.axis_index("c")
    s = lax.axis_index("s")
    w = s * NC + c
    sl = pl.ds(s * NPT, NPT)
    zd = pltpu.async_copy(zer_hbm.at[sl], acc.at[sl], sem_z)
    di = pltpu.async_copy(dst_hbm.at[w], idx_d, sem_i)
    od = pltpu.async_copy(ones_hbm, ones_v, sem_o)
    zd.wait()
    di.wait()
    od.wait()
    plsc.subcore_barrier()
    descs = []
    for j in range(NB1):
        descs.append(
            pltpu.async_copy(ones_v, acc.at[idx_d.at[j]], sem_sc, add=True))
    for d in descs:
        d.wait()
    plsc.subcore_barrier()
    pltpu.sync_copy(acc.at[sl], out_hbm.at[c, sl])


def _make_scatter(F, Bk, NBk):
    @functools.partial(
        pl.kernel,
        out_type=jax.ShapeDtypeStruct((NC, NP, F), jnp.float32),
        mesh=_mesh,
        scratch_types=[
            pltpu.VMEM_SHARED((NP, F), jnp.float32),
            pltpu.VMEM((NBk, Bk), jnp.int32),
            pltpu.VMEM((NBk, Bk), jnp.int32),
            pltpu.VMEM((Bk, F), jnp.float32),
            pltpu.VMEM((Bk, F), jnp.float32),
            pltpu.SemaphoreType.DMA,
            pltpu.SemaphoreType.DMA,
            pltpu.SemaphoreType.DMA,
            pltpu.SemaphoreType.DMA,
            pltpu.SemaphoreType.DMA,
            pltpu.SemaphoreType.DMA,
        ],
        compiler_params=_sc_params,
    )
    def _scatter(g_hbm, src_hbm, dst_hbm, zer_hbm, out_hbm, acc, idx_s, idx_d,
                 rows0, rows1, sem_z, sem_is, sem_id, sem_g, sem_sc0, sem_sc1):
        c = lax.axis_index("c")
        s = lax.axis_index("s")
        w = s * NC + c
        sl = pl.ds(s * NPT, NPT)
        zd = pltpu.async_copy(zer_hbm.at[sl], acc.at[sl], sem_z)
        sd = pltpu.async_copy(src_hbm.at[w], idx_s, sem_is)
        dd = pltpu.async_copy(dst_hbm.at[w], idx_d, sem_id)
        zd.wait()
        sd.wait()
        dd.wait()
        plsc.subcore_barrier()
        rows = (rows0, rows1)
        sem_sc = (sem_sc0, sem_sc1)
        scs = [None] * NBk
        for j in range(NBk):
            if j >= 2:
                scs[j - 2].wait()
            gd = pltpu.async_copy(g_hbm.at[idx_s.at[j]], rows[j % 2], sem_g)
            gd.wait()
            scs[j] = pltpu.async_copy(rows[j % 2], acc.at[idx_d.at[j]],
                                      sem_sc[j % 2], add=True)
        scs[NBk - 1].wait()
        scs[NBk - 2].wait()
        plsc.subcore_barrier()
        pltpu.sync_copy(acc.at[sl], out_hbm.at[c, sl])

    return _scatter


_scatter16 = _make_scatter(F1, B1, NB1)

EPS = E // NS
BP = 1000
NBP = EPS // BP


@functools.partial(
    pl.kernel,
    out_type=jax.ShapeDtypeStruct((NC, G, F1), jnp.float32),
    mesh=_mesh,
    scratch_types=[
        pltpu.VMEM_SHARED((NP, F1), jnp.float32),
        pltpu.VMEM_SHARED((NS, G + 1, F1), jnp.float32),
        pltpu.VMEM((NBP, BP), jnp.int32),
        pltpu.VMEM((NBP, BP), jnp.int32),
        pltpu.VMEM((BP, F1), jnp.float32),
        pltpu.VMEM((BP, F1), jnp.float32),
        pltpu.VMEM((NPT, F1), jnp.float32),
        pltpu.VMEM((NPT, F1), jnp.float32),
        pltpu.VMEM((NPT,), jnp.float32),
        pltpu.VMEM((NPT,), jnp.int32),
        pltpu.VMEM((F1,), jnp.float32),
        pltpu.VMEM((G + 1, F1), jnp.float32),
        pltpu.VMEM((NS, G + 1, F1), jnp.float32),
        pltpu.VMEM((F1,), jnp.float32),
        pltpu.VMEM((G, F1), jnp.float32),
        pltpu.SemaphoreType.DMA,
        pltpu.SemaphoreType.DMA,
        pltpu.SemaphoreType.DMA,
        pltpu.SemaphoreType.DMA,
        pltpu.SemaphoreType.DMA,
        pltpu.SemaphoreType.DMA,
        pltpu.SemaphoreType.DMA,
    ],
    compiler_params=_sc_params,
)
def _pool_kernel(g2f_hbm, src_hbm, dst_hbm, zer_hbm, dinv_hbm, b2_hbm,
                 bi_hbm, out_hbm, acc, pool_sh, idx_s, idx_d, rows0, rows1,
                 gst, ast, dvst, bist, b2v, poolbuf, allpool, cbuf, outbuf,
                 sem_z, sem_is, sem_id, sem_g, sem_sc0, sem_sc1, sem_m):
    c = lax.axis_index("c")
    s = lax.axis_index("s")
    sl = pl.ds(s * NPT, NPT)
    zd = pltpu.async_copy(zer_hbm.at[sl], acc.at[sl], sem_z)
    sd = pltpu.async_copy(src_hbm.at[c, s], idx_s, sem_is)
    dd = pltpu.async_copy(dst_hbm.at[s], idx_d, sem_id)
    zd.wait()
    sd.wait()
    dd.wait()
    plsc.subcore_barrier()
    rows = (rows0, rows1)
    sem_sc = (sem_sc0, sem_sc1)
    scs = [None] * NBP
    for j in range(NBP):
        if j >= 2:
            scs[j - 2].wait()
        gd = pltpu.async_copy(g2f_hbm.at[idx_s.at[j]], rows[j % 2], sem_g)
        gd.wait()
        scs[j] = pltpu.async_copy(rows[j % 2], acc.at[idx_d.at[j]],
                                  sem_sc[j % 2], add=True)
    scs[NBP - 1].wait()
    scs[NBP - 2].wait()
    plsc.subcore_barrier()
    d1 = pltpu.async_copy(g2f_hbm.at[pl.ds(c * NP + s * NPT, NPT)], gst,
                          sem_g)
    d2 = pltpu.async_copy(dinv_hbm.at[sl], dvst, sem_z)
    d3 = pltpu.async_copy(bi_hbm.at[s], bist, sem_is)
    d4 = pltpu.async_copy(b2_hbm.at[c], b2v, sem_id)
    pltpu.sync_copy(acc.at[sl], ast)
    d1.wait()
    d2.wait()
    d3.wait()
    d4.wait()

    zf = jnp.zeros((F1,), jnp.float32)
    one = jnp.ones((F1,), jnp.float32)
    lanef = lax.broadcasted_iota(jnp.int32, (F1,), 0).astype(jnp.float32)
    b2row = b2v[...]

    def group_body(i, carry):
        a0, a1, a2, a3, cnt = carry
        dv16 = dvst[pl.ds(i * 16, 16)]
        bf16 = bist[pl.ds(i * 16, 16)].astype(jnp.float32)
        for k in range(16):
            n = i * 16 + k
            dvec = zf + dv16[k]
            h2 = jnp.maximum(dvec * (ast[n, :] + gst[n, :]) + b2row, 0.0)
            bvec = zf + bf16[k]
            cnt = cnt + jnp.maximum(one - jnp.abs(lanef - bvec), 0.0)
            a0 = a0 + jnp.maximum(one - jnp.abs(bvec - 0.0), 0.0) * h2
            a1 = a1 + jnp.maximum(one - jnp.abs(bvec - 1.0), 0.0) * h2
            a2 = a2 + jnp.maximum(one - jnp.abs(bvec - 2.0), 0.0) * h2
            a3 = a3 + jnp.maximum(one - jnp.abs(bvec - 3.0), 0.0) * h2
        return (a0, a1, a2, a3, cnt)

    a0, a1, a2, a3, cnt = lax.fori_loop(0, NPT // 16, group_body,
                                        (zf, zf, zf, zf, zf))
    poolbuf[0, :] = a0
    poolbuf[1, :] = a1
    poolbuf[2, :] = a2
    poolbuf[3, :] = a3
    poolbuf[4, :] = cnt
    pltpu.sync_copy(poolbuf, pool_sh.at[s])
    plsc.subcore_barrier()

    @pl.when(s == 0)
    def _():
        pltpu.sync_copy(pool_sh, allpool)

        def tile_body(t, carry):
            s0, s1, s2, s3, sc = carry
            return (s0 + allpool[t, 0, :], s1 + allpool[t, 1, :],
                    s2 + allpool[t, 2, :], s3 + allpool[t, 3, :],
                    sc + allpool[t, 4, :])

        s0, s1, s2, s3, sc = lax.fori_loop(0, NS, tile_body,
                                           (zf, zf, zf, zf, zf))
        cbuf[...] = sc
        cv = cbuf[...]
        sums = (s0, s1, s2, s3)
        for g in range(G):
            cvec = zf + cv[g]
            outbuf[g, :] = sums[g] / jnp.maximum(cvec, 1.0)
        pltpu.sync_copy(outbuf, out_hbm.at[c])


RB = 1280
NRB = NP // RB


def _tc1_body(x_ref, w1_ref, dp_ref, g1_ref, dinv_ref):
    deg = dp_ref[0, :, 0:1] + dp_ref[1, :, 0:1] + 1.0
    dinv = lax.rsqrt(deg)
    hw = jnp.dot(x_ref[...], w1_ref[...],
                 preferred_element_type=jnp.float32,
                 precision=lax.Precision.HIGHEST)
    g1_ref[...] = hw * dinv
    dinv_ref[...] = dinv


def _tc1(xp, W1, degp):
    return pl.pallas_call(
        _tc1_body,
        grid=(NRB,),
        in_specs=[
            pl.BlockSpec((RB, D), lambda i: (i, 0)),
            pl.BlockSpec((D, F1), lambda i: (0, 0)),
            pl.BlockSpec((NC, RB, W8), lambda i: (0, i, 0)),
        ],
        out_specs=[
            pl.BlockSpec((RB, F1), lambda i: (i, 0)),
            pl.BlockSpec((RB, 1), lambda i: (i, 0)),
        ],
        out_shape=[
            jax.ShapeDtypeStruct((NP, F1), jnp.float32),
            jax.ShapeDtypeStruct((NP, 1), jnp.float32),
        ],
    )(xp, W1, degp)


def _tc2_body(sp_ref, g1_ref, dinv_ref, b1_ref, w2_ref, g2_ref):
    dv = dinv_ref[...]
    h1 = jnp.maximum(
        dv * (sp_ref[0] + sp_ref[1] + g1_ref[...]) + b1_ref[...], 0.0)
    g2 = jnp.dot(h1, w2_ref[...],
                 preferred_element_type=jnp.float32,
                 precision=lax.Precision.HIGHEST) * dv
    g2_ref[0] = g2[:, :F1]
    g2_ref[1] = g2[:, F1:]


def _tc2(sp, g1, dinv, b1r, W2):
    return pl.pallas_call(
        _tc2_body,
        grid=(NRB,),
        in_specs=[
            pl.BlockSpec((NC, RB, F1), lambda i: (0, i, 0)),
            pl.BlockSpec((RB, F1), lambda i: (i, 0)),
            pl.BlockSpec((RB, 1), lambda i: (i, 0)),
            pl.BlockSpec((1, F1), lambda i: (0, 0)),
            pl.BlockSpec((F1, F2), lambda i: (0, 0)),
        ],
        out_specs=pl.BlockSpec((NC, RB, F1), lambda i: (0, i, 0)),
        out_shape=jax.ShapeDtypeStruct((NC, NP, F1), jnp.float32),
    )(sp, g1, dinv, b1r, W2)


def kernel(x, edge_index, batch_index, W1, b1, W2, b2):
    x = x.astype(jnp.float32)
    src = edge_index[0].astype(jnp.int32)
    dst = edge_index[1].astype(jnp.int32)
    bi = batch_index.astype(jnp.int32)

    xp = jnp.pad(x, ((0, NP - N), (0, 0)))
    biR = jnp.pad(bi, (0, NP - N), constant_values=G).reshape(NS, NPT)

    src1 = src.reshape(NW, NB1, B1)
    dst1 = dst.reshape(NW, NB1, B1)
    srcO = jnp.stack([src, src + NP]).reshape(NC, NS, NBP, BP)
    dstP = dst.reshape(NS, NBP, BP)
    ones16 = jnp.ones((B1, W8), jnp.float32)
    z16 = jnp.zeros((NP, F1), jnp.float32)

    degp = _deg_kernel(dst1, ones16, z16)
    g1, dinv = _tc1(xp, W1, degp)
    sp = _scatter16(g1, src1, dst1, z16)
    g2s = _tc2(sp, g1, dinv, b1.reshape(1, F1), W2)
    g2f = g2s.reshape(NC * NP, F1)
    ph = _pool_kernel(g2f, srcO, dstP, z16, dinv.reshape(NP),
                      b2.reshape(NC, F1), biR)
    return jnp.concatenate([ph[0], ph[1]], axis=1)
